# baseline (speedup 1.0000x reference)
import functools

import jax
import jax.numpy as jnp
from jax import lax
from jax.experimental import pallas as pl
from jax.experimental.pallas import tpu as pltpu
from jax.experimental.pallas import tpu_sc as plsc

_G = 64
_C = 128
_NC, _NS = 2, 16
_NW = _NC * _NS
_BLK = 1024


def _msg_pass_sc(h, src, dst, attr, We, be, NP, EP):
    D = h.shape[1]
    ED = attr.shape[0] // EP
    PT = EP // _NW
    nchunks = PT // _C
    nwrite = NP // _NS
    mesh = plsc.VectorSubcoreMesh(core_axis_name="c", subcore_axis_name="s")

    @functools.partial(
        pl.kernel,
        out_type=(jax.ShapeDtypeStruct((NP, D), jnp.float32),
                  jax.ShapeDtypeStruct((NP, D), jnp.float32)),
        mesh=mesh,
        scratch_types=[
            pltpu.VMEM_SHARED((NP, D), jnp.float32),
            pltpu.VMEM((_C,), jnp.int32),
            pltpu.VMEM((_C,), jnp.int32),
            pltpu.VMEM((_C * ED + 16,), jnp.float32),
            pltpu.VMEM((_C, D), jnp.float32),
            pltpu.VMEM((ED, D), jnp.float32),
            pltpu.VMEM((1, D), jnp.float32),
            pltpu.SemaphoreType.DMA,
        ],
    )
    def k(h_hbm, src_hbm, dst_hbm, attr_hbm, we_hbm, be_hbm,
          out0, out1, acc, si, di, av, rows, wv, bv, sem):
        cid = lax.axis_index("c")
        sid = lax.axis_index("s")
        wid = cid * _NS + sid

        def zrow(r, carry):
            for j in range(D // 16):
                rows[r, pl.ds(j * 16, 16)] = jnp.zeros((16,), jnp.float32)
            return carry
        lax.fori_loop(0, _C, zrow, 0)
        for i in range(nwrite // _C):
            pltpu.sync_copy(rows, acc.at[pl.ds(sid * nwrite + i * _C, _C)])

        pltpu.sync_copy(we_hbm, wv)
        pltpu.sync_copy(be_hbm, bv)
        plsc.subcore_barrier()

        def chunk(i, carry):
            base = wid * PT + i * _C
            pltpu.sync_copy(src_hbm.at[pl.ds(base, _C)], si)
            pltpu.sync_copy(dst_hbm.at[pl.ds(base, _C)], di)
            pltpu.sync_copy(attr_hbm.at[pl.ds(base * ED, _C * ED)],
                            av.at[pl.ds(0, _C * ED)])
            pltpu.async_copy(h_hbm.at[si], rows, sem).wait()

            def edge(c, ecarry):
                avec = av[pl.ds(c * ED, 16)]
                a = [avec[kk] for kk in range(ED)]
                for j in range(D // 16):
                    sl = pl.ds(j * 16, 16)
                    e = a[0] * wv[0, sl]
                    for kk in range(1, ED):
                        e = e + a[kk] * wv[kk, sl]
                    t = rows[c, sl] + (e + bv[0, sl])
                    rows[c, sl] = jnp.maximum(t, 0.0)
                return ecarry
            lax.fori_loop(0, _C, edge, 0)
            pltpu.sync_copy(rows, acc.at[di], add=True)
            return carry
        lax.fori_loop(0, nchunks, chunk, 0)

        plsc.subcore_barrier()

        @pl.when(cid == 0)
        def _():
            pltpu.sync_copy(acc.at[pl.ds(sid * nwrite, nwrite)],
                            out0.at[pl.ds(sid * nwrite, nwrite)])

        @pl.when(cid == 1)
        def _():
            pltpu.sync_copy(acc.at[pl.ds(sid * nwrite, nwrite)],
                            out1.at[pl.ds(sid * nwrite, nwrite)])

    return k(h, src, dst, attr, We, be.reshape(1, D))


def _gine_mlp(x, a0, a1, Wa, ba, Wb, bb):
    NP, D = x.shape
    H = Wa.shape[1]

    def body(x_ref, a0_ref, a1_ref, wa_ref, ba_ref, wb_ref, bb_ref, o_ref):
        z = x_ref[...] + a0_ref[...] + a1_ref[...]
        t = jnp.maximum(
            jnp.dot(z.astype(jnp.bfloat16), wa_ref[...].astype(jnp.bfloat16),
                    preferred_element_type=jnp.float32)
            + ba_ref[...], 0.0)
        o_ref[...] = jnp.maximum(
            jnp.dot(t.astype(jnp.bfloat16), wb_ref[...].astype(jnp.bfloat16),
                    preferred_element_type=jnp.float32)
            + bb_ref[...], 0.0)

    return pl.pallas_call(
        body,
        grid=(NP // _BLK,),
        in_specs=[pl.BlockSpec((_BLK, D), lambda i: (i, 0))] * 3 + [
            pl.BlockSpec((D, H), lambda i: (0, 0)),
            pl.BlockSpec((1, H), lambda i: (0, 0)),
            pl.BlockSpec((H, H), lambda i: (0, 0)),
            pl.BlockSpec((1, H), lambda i: (0, 0)),
        ],
        out_specs=pl.BlockSpec((_BLK, H), lambda i: (i, 0)),
        out_shape=jax.ShapeDtypeStruct((NP, H), jnp.float32),
    )(x, a0, a1, Wa, ba.reshape(1, H), Wb, bb.reshape(1, H))


def _gine_mlp_pool(h, a0, a1, batch3, Wa, ba, Wb, bb):
    NP, D = h.shape
    H = Wa.shape[1]
    NB = NP // _BLK

    def body(h_ref, a0_ref, a1_ref, b3_ref, wa_ref, ba_ref, wb_ref, bb_ref,
             sums_ref, cnt_ref):
        i = pl.program_id(0)
        z = h_ref[...] + a0_ref[...] + a1_ref[...]
        t = jnp.maximum(
            jnp.dot(z.astype(jnp.bfloat16), wa_ref[...].astype(jnp.bfloat16),
                    preferred_element_type=jnp.float32)
            + ba_ref[...], 0.0)
        h2 = jnp.maximum(
            jnp.dot(t.astype(jnp.bfloat16), wb_ref[...].astype(jnp.bfloat16),
                    preferred_element_type=jnp.float32)
            + bb_ref[...], 0.0)
        b = b3_ref[0, 0, :]
        oh = (b[:, None] == lax.broadcasted_iota(jnp.int32, (_BLK, _G), 1)
              ).astype(jnp.float32)
        psum = lax.dot_general(oh, h2, (((0,), (0,)), ((), ())),
                               preferred_element_type=jnp.float32,
                    precision=lax.Precision.HIGHEST)
        pcnt = jnp.sum(oh, axis=0)[:, None]

        @pl.when(i == 0)
        def _():
            sums_ref[...] = jnp.zeros_like(sums_ref)
            cnt_ref[...] = jnp.zeros_like(cnt_ref)

        sums_ref[...] += psum
        cnt_ref[...] += pcnt

    return pl.pallas_call(
        body,
        grid=(NB,),
        in_specs=[pl.BlockSpec((_BLK, D), lambda i: (i, 0))] * 3 + [
            pl.BlockSpec((1, 1, _BLK), lambda i: (i, 0, 0)),
            pl.BlockSpec((D, H), lambda i: (0, 0)),
            pl.BlockSpec((1, H), lambda i: (0, 0)),
            pl.BlockSpec((H, H), lambda i: (0, 0)),
            pl.BlockSpec((1, H), lambda i: (0, 0)),
        ],
        out_specs=(pl.BlockSpec((_G, H), lambda i: (0, 0)),
                   pl.BlockSpec((_G, 1), lambda i: (0, 0))),
        out_shape=(jax.ShapeDtypeStruct((_G, H), jnp.float32),
                   jax.ShapeDtypeStruct((_G, 1), jnp.float32)),
    )(h, a0, a1, batch3, Wa, ba.reshape(1, H), Wb, bb.reshape(1, H))


def _classifier(sums, cnt, Wc1, bc1, Wc2, bc2):
    H = Wc1.shape[0]

    def body(s_ref, c_ref, w1_ref, b1_ref, w2_ref, b2_ref, o_ref):
        pooled = s_ref[...] / jnp.maximum(c_ref[...], 1.0)
        hid = jnp.maximum(
            jnp.dot(pooled.astype(jnp.bfloat16), w1_ref[...].astype(jnp.bfloat16),
                    preferred_element_type=jnp.float32)
            + b1_ref[...], 0.0)
        o_ref[...] = (jnp.dot(hid.astype(jnp.bfloat16),
                              w2_ref[...].astype(jnp.bfloat16),
                              preferred_element_type=jnp.float32)
                      + b2_ref[...])

    return pl.pallas_call(
        body,
        out_shape=jax.ShapeDtypeStruct((_G, 1), jnp.float32),
    )(sums, cnt, Wc1, bc1.reshape(1, H), Wc2, bc2.reshape(1, 1))


def kernel(x, edge_index, edge_attr, batch, We1, be1, W1, b1, W2, b2,
           We2, be2, W3, b3, W4, b4, Wc1, bc1, Wc2, bc2):
    N, D = x.shape
    E = edge_index.shape[1]

    NP = -(-N // 2048) * 2048
    EP = -(-E // (_NW * _C)) * (_NW * _C)

    x_pad = jnp.pad(x, ((0, NP - N), (0, 0)))
    src = jnp.pad(edge_index[0].astype(jnp.int32), (0, EP - E))
    dst = jnp.pad(edge_index[1].astype(jnp.int32), (0, EP - E),
                  constant_values=N)
    attr_r = edge_attr.astype(jnp.bfloat16).astype(jnp.float32)
    We1r = We1.astype(jnp.bfloat16).astype(jnp.float32)
    We2r = We2.astype(jnp.bfloat16).astype(jnp.float32)
    attr_p = jnp.pad(attr_r, ((0, EP - E), (0, 0))).reshape(-1)
    batch3 = jnp.pad(batch.astype(jnp.int32), (0, NP - N),
                     constant_values=_G).reshape(NP // _BLK, 1, _BLK)

    a0, a1 = _msg_pass_sc(x_pad, src, dst, attr_p, We1r, be1, NP, EP)
    h1 = _gine_mlp(x_pad, a0, a1, W1, b1, W2, b2)
    c0, c1 = _msg_pass_sc(h1, src, dst, attr_p, We2r, be2, NP, EP)
    sums, cnt = _gine_mlp_pool(h1, c0, c1, batch3, W3, b3, W4, b4)
    return _classifier(sums, cnt, Wc1, bc1, Wc2, bc2)


# speedup vs baseline: 1.8466x; 1.8466x over previous
import functools

import jax
import jax.numpy as jnp
from jax import lax
from jax.experimental import pallas as pl
from jax.experimental.pallas import tpu as pltpu
from jax.experimental.pallas import tpu_sc as plsc

_G = 64
_C = 128
_NC, _NS = 2, 16
_NW = _NC * _NS
_BLK = 1024


def _msg_pass_sc(h, src, dst, attr, We, be, NP, EP):
    D = h.shape[1]
    ED = attr.shape[0] // EP
    PT = EP // _NW
    nchunks = PT // _C
    nwrite = NP // _NS
    mesh = plsc.VectorSubcoreMesh(core_axis_name="c", subcore_axis_name="s")

    @functools.partial(
        pl.kernel,
        out_type=(jax.ShapeDtypeStruct((NP, D), jnp.float32),
                  jax.ShapeDtypeStruct((NP, D), jnp.float32)),
        mesh=mesh,
        scratch_types=[
            pltpu.VMEM_SHARED((NP, D), jnp.float32),
            pltpu.VMEM((_C,), jnp.int32),
            pltpu.VMEM((_C,), jnp.int32),
            pltpu.VMEM((_C * ED + 16,), jnp.float32),
            pltpu.VMEM((_C, D), jnp.float32),
            pltpu.VMEM((ED, D), jnp.float32),
            pltpu.VMEM((1, D), jnp.float32),
            pltpu.SemaphoreType.DMA,
        ],
    )
    def k(h_hbm, src_hbm, dst_hbm, attr_hbm, we_hbm, be_hbm,
          out0, out1, acc, si, di, av, rows, wv, bv, sem):
        cid = lax.axis_index("c")
        sid = lax.axis_index("s")
        wid = cid * _NS + sid

        def zrow(r, carry):
            for j in range(D // 16):
                rows[r, pl.ds(j * 16, 16)] = jnp.zeros((16,), jnp.float32)
            return carry
        lax.fori_loop(0, _C, zrow, 0)
        for i in range(nwrite // _C):
            pltpu.sync_copy(rows, acc.at[pl.ds(sid * nwrite + i * _C, _C)])

        pltpu.sync_copy(we_hbm, wv)
        pltpu.sync_copy(be_hbm, bv)
        plsc.subcore_barrier()

        def chunk(i, carry):
            wvec = [[wv[kk, pl.ds(j * 16, 16)] for j in range(D // 16)]
                    for kk in range(ED)]
            bvec = [bv[0, pl.ds(j * 16, 16)] for j in range(D // 16)]
            base = wid * PT + i * _C
            pltpu.sync_copy(src_hbm.at[pl.ds(base, _C)], si)
            pltpu.sync_copy(dst_hbm.at[pl.ds(base, _C)], di)
            pltpu.sync_copy(attr_hbm.at[pl.ds(base * ED, _C * ED)],
                            av.at[pl.ds(0, _C * ED)])
            pltpu.async_copy(h_hbm.at[si], rows, sem).wait()

            def edge(c, ecarry):
                avec = av[pl.ds(c * ED, 16)]
                a = [avec[kk] for kk in range(ED)]
                for j in range(D // 16):
                    sl = pl.ds(j * 16, 16)
                    e = a[0] * wvec[0][j]
                    for kk in range(1, ED):
                        e = e + a[kk] * wvec[kk][j]
                    t = rows[c, sl] + (e + bvec[j])
                    rows[c, sl] = jnp.maximum(t, 0.0)
                return ecarry
            lax.fori_loop(0, _C, edge, 0)
            pltpu.sync_copy(rows, acc.at[di], add=True)
            return carry
        lax.fori_loop(0, nchunks, chunk, 0)

        plsc.subcore_barrier()

        @pl.when(cid == 0)
        def _():
            pltpu.sync_copy(acc.at[pl.ds(sid * nwrite, nwrite)],
                            out0.at[pl.ds(sid * nwrite, nwrite)])

        @pl.when(cid == 1)
        def _():
            pltpu.sync_copy(acc.at[pl.ds(sid * nwrite, nwrite)],
                            out1.at[pl.ds(sid * nwrite, nwrite)])

    return k(h, src, dst, attr, We, be.reshape(1, D))


def _gine_mlp(x, a0, a1, Wa, ba, Wb, bb):
    NP, D = x.shape
    H = Wa.shape[1]

    def body(x_ref, a0_ref, a1_ref, wa_ref, ba_ref, wb_ref, bb_ref, o_ref):
        z = x_ref[...] + a0_ref[...] + a1_ref[...]
        t = jnp.maximum(
            jnp.dot(z.astype(jnp.bfloat16), wa_ref[...].astype(jnp.bfloat16),
                    preferred_element_type=jnp.float32)
            + ba_ref[...], 0.0)
        o_ref[...] = jnp.maximum(
            jnp.dot(t.astype(jnp.bfloat16), wb_ref[...].astype(jnp.bfloat16),
                    preferred_element_type=jnp.float32)
            + bb_ref[...], 0.0)

    return pl.pallas_call(
        body,
        grid=(NP // _BLK,),
        in_specs=[pl.BlockSpec((_BLK, D), lambda i: (i, 0))] * 3 + [
            pl.BlockSpec((D, H), lambda i: (0, 0)),
            pl.BlockSpec((1, H), lambda i: (0, 0)),
            pl.BlockSpec((H, H), lambda i: (0, 0)),
            pl.BlockSpec((1, H), lambda i: (0, 0)),
        ],
        out_specs=pl.BlockSpec((_BLK, H), lambda i: (i, 0)),
        out_shape=jax.ShapeDtypeStruct((NP, H), jnp.float32),
    )(x, a0, a1, Wa, ba.reshape(1, H), Wb, bb.reshape(1, H))


def _gine_mlp_pool(h, a0, a1, batch3, Wa, ba, Wb, bb):
    NP, D = h.shape
    H = Wa.shape[1]
    NB = NP // _BLK

    def body(h_ref, a0_ref, a1_ref, b3_ref, wa_ref, ba_ref, wb_ref, bb_ref,
             sums_ref, cnt_ref):
        i = pl.program_id(0)
        z = h_ref[...] + a0_ref[...] + a1_ref[...]
        t = jnp.maximum(
            jnp.dot(z.astype(jnp.bfloat16), wa_ref[...].astype(jnp.bfloat16),
                    preferred_element_type=jnp.float32)
            + ba_ref[...], 0.0)
        h2 = jnp.maximum(
            jnp.dot(t.astype(jnp.bfloat16), wb_ref[...].astype(jnp.bfloat16),
                    preferred_element_type=jnp.float32)
            + bb_ref[...], 0.0)
        b = b3_ref[0, 0, :]
        oh = (b[:, None] == lax.broadcasted_iota(jnp.int32, (_BLK, _G), 1)
              ).astype(jnp.float32)
        psum = lax.dot_general(oh, h2, (((0,), (0,)), ((), ())),
                               preferred_element_type=jnp.float32,
                    precision=lax.Precision.HIGHEST)
        pcnt = jnp.sum(oh, axis=0)[:, None]

        @pl.when(i == 0)
        def _():
            sums_ref[...] = jnp.zeros_like(sums_ref)
            cnt_ref[...] = jnp.zeros_like(cnt_ref)

        sums_ref[...] += psum
        cnt_ref[...] += pcnt

    return pl.pallas_call(
        body,
        grid=(NB,),
        in_specs=[pl.BlockSpec((_BLK, D), lambda i: (i, 0))] * 3 + [
            pl.BlockSpec((1, 1, _BLK), lambda i: (i, 0, 0)),
            pl.BlockSpec((D, H), lambda i: (0, 0)),
            pl.BlockSpec((1, H), lambda i: (0, 0)),
            pl.BlockSpec((H, H), lambda i: (0, 0)),
            pl.BlockSpec((1, H), lambda i: (0, 0)),
        ],
        out_specs=(pl.BlockSpec((_G, H), lambda i: (0, 0)),
                   pl.BlockSpec((_G, 1), lambda i: (0, 0))),
        out_shape=(jax.ShapeDtypeStruct((_G, H), jnp.float32),
                   jax.ShapeDtypeStruct((_G, 1), jnp.float32)),
    )(h, a0, a1, batch3, Wa, ba.reshape(1, H), Wb, bb.reshape(1, H))


def _classifier(sums, cnt, Wc1, bc1, Wc2, bc2):
    H = Wc1.shape[0]

    def body(s_ref, c_ref, w1_ref, b1_ref, w2_ref, b2_ref, o_ref):
        pooled = s_ref[...] / jnp.maximum(c_ref[...], 1.0)
        hid = jnp.maximum(
            jnp.dot(pooled.astype(jnp.bfloat16), w1_ref[...].astype(jnp.bfloat16),
                    preferred_element_type=jnp.float32)
            + b1_ref[...], 0.0)
        o_ref[...] = (jnp.dot(hid.astype(jnp.bfloat16),
                              w2_ref[...].astype(jnp.bfloat16),
                              preferred_element_type=jnp.float32)
                      + b2_ref[...])

    return pl.pallas_call(
        body,
        out_shape=jax.ShapeDtypeStruct((_G, 1), jnp.float32),
    )(sums, cnt, Wc1, bc1.reshape(1, H), Wc2, bc2.reshape(1, 1))


def kernel(x, edge_index, edge_attr, batch, We1, be1, W1, b1, W2, b2,
           We2, be2, W3, b3, W4, b4, Wc1, bc1, Wc2, bc2):
    N, D = x.shape
    E = edge_index.shape[1]

    NP = -(-N // 2048) * 2048
    EP = -(-E // (_NW * _C)) * (_NW * _C)

    x_pad = jnp.pad(x, ((0, NP - N), (0, 0)))
    src = jnp.pad(edge_index[0].astype(jnp.int32), (0, EP - E))
    dst = jnp.pad(edge_index[1].astype(jnp.int32), (0, EP - E),
                  constant_values=N)
    attr_r = edge_attr.astype(jnp.bfloat16).astype(jnp.float32)
    We1r = We1.astype(jnp.bfloat16).astype(jnp.float32)
    We2r = We2.astype(jnp.bfloat16).astype(jnp.float32)
    attr_p = jnp.pad(attr_r, ((0, EP - E), (0, 0))).reshape(-1)
    batch3 = jnp.pad(batch.astype(jnp.int32), (0, NP - N),
                     constant_values=_G).reshape(NP // _BLK, 1, _BLK)

    a0, a1 = _msg_pass_sc(x_pad, src, dst, attr_p, We1r, be1, NP, EP)
    h1 = _gine_mlp(x_pad, a0, a1, W1, b1, W2, b2)
    c0, c1 = _msg_pass_sc(h1, src, dst, attr_p, We2r, be2, NP, EP)
    sums, cnt = _gine_mlp_pool(h1, c0, c1, batch3, W3, b3, W4, b4)
    return _classifier(sums, cnt, Wc1, bc1, Wc2, bc2)
